# Initial kernel scaffold; baseline (speedup 1.0000x reference)
import jax
import jax.numpy as jnp
from jax.experimental import pallas as pl


def kernel(x, edge_index, W0, b0, W1, b1):
    raise NotImplementedError("write your pallas kernel here")


# speedup vs baseline: 15.9707x; 15.9707x over previous
import functools

import jax
import jax.numpy as jnp
from jax import lax
from jax.experimental import pallas as pl
from jax.experimental.pallas import tpu as pltpu
from jax.experimental.pallas import tpu_sc as plsc

N = 10000
F = 128
E = 320000
NC = 2
NS = 16
T = NC * NS
EPT = E // T
B = 128
NB = -(-EPT // B)
EPAD = NB * B
NP = 10240
ROWS_PT = NP // NS
CH = ROWS_PT // 4
DF = 16
HF = F // 2
EPT2 = E // NS
NB2 = -(-EPT2 // B)
EPAD2 = NB2 * B
BLK = 1000


def _mesh():
    return plsc.VectorSubcoreMesh(core_axis_name="c", subcore_axis_name="s",
                                  num_cores=NC, num_subcores=NS)


@functools.cache
def _sc_degree_kernel():
    @functools.partial(
        pl.kernel,
        out_type=jax.ShapeDtypeStruct((NC, NP, DF), jnp.float32),
        mesh=_mesh(),
        scratch_types=[
            pltpu.VMEM((NB, B), jnp.int32),
            pltpu.VMEM((B, DF), jnp.float32),
            pltpu.VMEM((ROWS_PT, DF), jnp.float32),
            pltpu.VMEM_SHARED((NP, DF), jnp.float32),
        ],
        compiler_params=pltpu.CompilerParams(use_tc_tiling_on_sc=False),
    )
    def _sc_degree(dstp_hbm, ones_hbm, z_hbm, out_hbm, dst_v, ones_v, ch_v, dacc):
        cid = lax.axis_index("c")
        sid = lax.axis_index("s")
        base = sid * ROWS_PT
        pltpu.sync_copy(z_hbm, ch_v)
        pltpu.sync_copy(ch_v, dacc.at[pl.ds(base, ROWS_PT)])
        pltpu.sync_copy(ones_hbm, ones_v)
        pltpu.sync_copy(dstp_hbm.at[cid, sid], dst_v)
        plsc.subcore_barrier()

        def body(j, carry):
            pltpu.sync_copy(ones_v, dacc.at[dst_v.at[j]], add=True)
            return carry

        lax.fori_loop(0, NB, body, 0)
        plsc.subcore_barrier()
        pltpu.sync_copy(dacc.at[pl.ds(base, ROWS_PT)], ch_v)
        pltpu.sync_copy(ch_v, out_hbm.at[cid, pl.ds(base, ROWS_PT)])

    return _sc_degree


@functools.cache
def _sc_scatter_kernel():
    @functools.partial(
        pl.kernel,
        out_type=jax.ShapeDtypeStruct((NC, NP, HF), jnp.float32),
        mesh=_mesh(),
        scratch_types=[
            pltpu.VMEM((NB2, B), jnp.int32),
            pltpu.VMEM((NB2, B), jnp.int32),
            pltpu.VMEM((B, HF), jnp.float32),
            pltpu.VMEM((CH, HF), jnp.float32),
            pltpu.VMEM_SHARED((NP, HF), jnp.float32),
            pltpu.SemaphoreType.DMA,
        ],
        compiler_params=pltpu.CompilerParams(use_tc_tiling_on_sc=False),
    )
    def _sc_scatter(g_hbm, srcp_hbm, dstp_hbm, z_hbm, out_hbm,
                    src_v, dst_v, rows_v, ch_v, acc, sem):
        cid = lax.axis_index("c")
        sid = lax.axis_index("s")
        base = sid * ROWS_PT
        pltpu.sync_copy(z_hbm, ch_v)
        for k in range(ROWS_PT // CH):
            pltpu.sync_copy(ch_v, acc.at[pl.ds(base + k * CH, CH)])
        pltpu.sync_copy(srcp_hbm.at[cid, sid], src_v)
        pltpu.sync_copy(dstp_hbm.at[sid], dst_v)
        plsc.subcore_barrier()

        def body(j, carry):
            pltpu.async_copy(g_hbm.at[src_v.at[j]], rows_v, sem).wait()
            pltpu.sync_copy(rows_v, acc.at[dst_v.at[j]], add=True)
            return carry

        lax.fori_loop(0, NB2, body, 0)
        plsc.subcore_barrier()
        for k in range(ROWS_PT // CH):
            pltpu.sync_copy(acc.at[pl.ds(base + k * CH, CH)], ch_v)
            pltpu.sync_copy(ch_v, out_hbm.at[cid, pl.ds(base + k * CH, CH)])

    return _sc_scatter


def _d_of(dp_ref):
    return lax.rsqrt(dp_ref[0, :, :1] + dp_ref[1, :, :1] + 1.0)


def _split_cols(v):
    return jnp.stack([v[:, :HF], v[:, HF:]], axis=0)


def _tc_layer0(x, W0, degp):
    def body(x_ref, w_ref, dp_ref, o_ref):
        d = _d_of(dp_ref)
        res = d * jnp.dot(x_ref[...], w_ref[...],
                          preferred_element_type=jnp.float32)
        o_ref[0] = res[:, :HF]
        o_ref[1] = res[:, HF:]

    return pl.pallas_call(
        body,
        grid=(N // BLK,),
        in_specs=[
            pl.BlockSpec((BLK, F), lambda i: (i, 0)),
            pl.BlockSpec((F, F), lambda i: (0, 0)),
            pl.BlockSpec((2, BLK, DF), lambda i: (0, i, 0)),
        ],
        out_specs=pl.BlockSpec((2, BLK, HF), lambda i: (0, i, 0)),
        out_shape=jax.ShapeDtypeStruct((2, N, HF), jnp.float32),
    )(x, W0, degp)


def _tc_layer1(p, g0, degp, b0, W1):
    def body(p_ref, g_ref, dp_ref, b_ref, w_ref, o_ref):
        d = _d_of(dp_ref)
        s = jnp.concatenate([p_ref[0] + g_ref[0], p_ref[1] + g_ref[1]],
                            axis=-1)
        h = jnp.maximum(d * s + b_ref[...], 0.0)
        res = d * jnp.dot(h, w_ref[...], preferred_element_type=jnp.float32)
        o_ref[0] = res[:, :HF]
        o_ref[1] = res[:, HF:]

    return pl.pallas_call(
        body,
        grid=(N // BLK,),
        in_specs=[
            pl.BlockSpec((2, BLK, HF), lambda i: (0, i, 0)),
            pl.BlockSpec((2, BLK, HF), lambda i: (0, i, 0)),
            pl.BlockSpec((2, BLK, DF), lambda i: (0, i, 0)),
            pl.BlockSpec((1, F), lambda i: (0, 0)),
            pl.BlockSpec((F, F), lambda i: (0, 0)),
        ],
        out_specs=pl.BlockSpec((2, BLK, HF), lambda i: (0, i, 0)),
        out_shape=jax.ShapeDtypeStruct((2, N, HF), jnp.float32),
    )(p, g0, degp, b0, W1)


def _tc_layer2(q, g1, degp, b1):
    def body(q_ref, g_ref, dp_ref, b_ref, o_ref):
        d = _d_of(dp_ref)
        s = jnp.concatenate([q_ref[0] + g_ref[0], q_ref[1] + g_ref[1]],
                            axis=-1)
        o_ref[...] = jax.nn.sigmoid(d * s + b_ref[...])

    return pl.pallas_call(
        body,
        grid=(N // BLK,),
        in_specs=[
            pl.BlockSpec((2, BLK, HF), lambda i: (0, i, 0)),
            pl.BlockSpec((2, BLK, HF), lambda i: (0, i, 0)),
            pl.BlockSpec((2, BLK, DF), lambda i: (0, i, 0)),
            pl.BlockSpec((1, F), lambda i: (0, 0)),
        ],
        out_specs=pl.BlockSpec((BLK, F), lambda i: (i, 0)),
        out_shape=jax.ShapeDtypeStruct((N, F), jnp.float32),
    )(q, g1, degp, b1)


def kernel(x, edge_index, W0, b0, W1, b1):
    src = edge_index[0]
    dst = edge_index[1]
    dpad = EPAD - EPT
    dstp_deg = jnp.pad(dst.reshape(T, EPT), ((0, 0), (0, dpad)),
                       constant_values=N).reshape(NC, NS, NB, B)
    spad = EPAD2 - EPT2
    src_t = jnp.pad(src.reshape(NS, EPT2),
                    ((0, 0), (0, spad))).reshape(NS, NB2, B)
    srcp = jnp.stack([src_t, src_t + N], axis=0)
    dstp = jnp.pad(dst.reshape(NS, EPT2), ((0, 0), (0, spad)),
                   constant_values=N).reshape(NS, NB2, B)
    ones = jnp.ones((B, DF), jnp.float32)
    zdeg = jnp.zeros((ROWS_PT, DF), jnp.float32)
    zrow = jnp.zeros((CH, HF), jnp.float32)

    degp = _sc_degree_kernel()(dstp_deg, ones, zdeg)
    g0 = _tc_layer0(x, W0, degp)
    p = _sc_scatter_kernel()(g0.reshape(NC * N, HF), srcp, dstp, zrow)
    g1 = _tc_layer1(p, g0, degp, b0.reshape(1, F), W1)
    q = _sc_scatter_kernel()(g1.reshape(NC * N, HF), srcp, dstp, zrow)
    return _tc_layer2(q, g1, degp, b1.reshape(1, F))


# speedup vs baseline: 22.8412x; 1.4302x over previous
import functools

import jax
import jax.numpy as jnp
from jax import lax
from jax.experimental import pallas as pl
from jax.experimental.pallas import tpu as pltpu
from jax.experimental.pallas import tpu_sc as plsc

N = 10000
F = 128
E = 320000
NC = 2
NS = 16
T = NC * NS
EPT = E // T
B = 128
NB = -(-EPT // B)
EPAD = NB * B
NP = 10240
ROWS_PT = NP // NS
CH = ROWS_PT // 4
DF = 16
HF = F // 2
EPT2 = E // NS
NB2 = -(-EPT2 // B)
EPAD2 = NB2 * B
BLK = 1000


def _mesh():
    return plsc.VectorSubcoreMesh(core_axis_name="c", subcore_axis_name="s",
                                  num_cores=NC, num_subcores=NS)


@functools.cache
def _sc_degree_kernel():
    @functools.partial(
        pl.kernel,
        out_type=jax.ShapeDtypeStruct((NC, NP, DF), jnp.float32),
        mesh=_mesh(),
        scratch_types=[
            pltpu.VMEM((NB, B), jnp.int32),
            pltpu.VMEM((B, DF), jnp.float32),
            pltpu.VMEM((ROWS_PT, DF), jnp.float32),
            pltpu.VMEM_SHARED((NP, DF), jnp.float32),
        ],
        compiler_params=pltpu.CompilerParams(use_tc_tiling_on_sc=False),
    )
    def _sc_degree(dstp_hbm, ones_hbm, z_hbm, out_hbm, dst_v, ones_v, ch_v, dacc):
        cid = lax.axis_index("c")
        sid = lax.axis_index("s")
        base = sid * ROWS_PT
        pltpu.sync_copy(z_hbm, ch_v)
        pltpu.sync_copy(ch_v, dacc.at[pl.ds(base, ROWS_PT)])
        pltpu.sync_copy(ones_hbm, ones_v)
        pltpu.sync_copy(dstp_hbm.at[cid, sid], dst_v)
        plsc.subcore_barrier()

        def body(j, carry):
            pltpu.sync_copy(ones_v, dacc.at[dst_v.at[j]], add=True)
            return carry

        lax.fori_loop(0, NB, body, 0)
        plsc.subcore_barrier()
        pltpu.sync_copy(dacc.at[pl.ds(base, ROWS_PT)], ch_v)
        pltpu.sync_copy(ch_v, out_hbm.at[cid, pl.ds(base, ROWS_PT)])

    return _sc_degree


@functools.cache
def _sc_scatter_kernel():
    @functools.partial(
        pl.kernel,
        out_type=jax.ShapeDtypeStruct((NC, NP, HF), jnp.float32),
        mesh=_mesh(),
        scratch_types=[
            pltpu.VMEM((NB2, B), jnp.int32),
            pltpu.VMEM((NB2, B), jnp.int32),
            pltpu.VMEM((B, HF), jnp.float32),
            pltpu.VMEM((B, HF), jnp.float32),
            pltpu.VMEM((CH, HF), jnp.float32),
            pltpu.VMEM_SHARED((NP, HF), jnp.float32),
            pltpu.SemaphoreType.DMA,
            pltpu.SemaphoreType.DMA,
        ],
        compiler_params=pltpu.CompilerParams(use_tc_tiling_on_sc=False),
    )
    def _sc_scatter(g_hbm, srcp_hbm, dstp_hbm, z_hbm, out_hbm,
                    src_v, dst_v, rows0_v, rows1_v, ch_v, acc, sem0, sem1):
        cid = lax.axis_index("c")
        sid = lax.axis_index("s")
        base = sid * ROWS_PT
        pltpu.sync_copy(z_hbm, ch_v)
        for k in range(ROWS_PT // CH):
            pltpu.sync_copy(ch_v, acc.at[pl.ds(base + k * CH, CH)])
        pltpu.sync_copy(srcp_hbm.at[cid, sid], src_v)
        pltpu.sync_copy(dstp_hbm.at[sid], dst_v)
        plsc.subcore_barrier()

        d0 = pltpu.async_copy(g_hbm.at[src_v.at[0]], rows0_v, sem0)

        def body(i, carry):
            j0 = 2 * i
            d1 = pltpu.async_copy(g_hbm.at[src_v.at[j0 + 1]], rows1_v, sem1)
            d0.wait()
            pltpu.sync_copy(rows0_v, acc.at[dst_v.at[j0]], add=True)
            pltpu.async_copy(g_hbm.at[src_v.at[j0 + 2]], rows0_v, sem0)
            d1.wait()
            pltpu.sync_copy(rows1_v, acc.at[dst_v.at[j0 + 1]], add=True)
            return carry

        lax.fori_loop(0, NB2 // 2, body, 0)
        d0.wait()
        pltpu.sync_copy(rows0_v, acc.at[dst_v.at[NB2 - 1]], add=True)
        plsc.subcore_barrier()
        for k in range(ROWS_PT // CH):
            pltpu.sync_copy(acc.at[pl.ds(base + k * CH, CH)], ch_v)
            pltpu.sync_copy(ch_v, out_hbm.at[cid, pl.ds(base + k * CH, CH)])

    return _sc_scatter


def _d_of(dp_ref):
    return lax.rsqrt(dp_ref[0, :, :1] + dp_ref[1, :, :1] + 1.0)


def _split_cols(v):
    return jnp.stack([v[:, :HF], v[:, HF:]], axis=0)


def _tc_layer0(x, W0, degp):
    def body(x_ref, w_ref, dp_ref, o_ref):
        d = _d_of(dp_ref)
        res = d * jnp.dot(x_ref[...], w_ref[...],
                          preferred_element_type=jnp.float32)
        o_ref[0] = res[:, :HF]
        o_ref[1] = res[:, HF:]

    return pl.pallas_call(
        body,
        grid=(N // BLK,),
        in_specs=[
            pl.BlockSpec((BLK, F), lambda i: (i, 0)),
            pl.BlockSpec((F, F), lambda i: (0, 0)),
            pl.BlockSpec((2, BLK, DF), lambda i: (0, i, 0)),
        ],
        out_specs=pl.BlockSpec((2, BLK, HF), lambda i: (0, i, 0)),
        out_shape=jax.ShapeDtypeStruct((2, N, HF), jnp.float32),
    )(x, W0, degp)


def _tc_layer1(p, g0, degp, b0, W1):
    def body(p_ref, g_ref, dp_ref, b_ref, w_ref, o_ref):
        d = _d_of(dp_ref)
        s = jnp.concatenate([p_ref[0] + g_ref[0], p_ref[1] + g_ref[1]],
                            axis=-1)
        h = jnp.maximum(d * s + b_ref[...], 0.0)
        res = d * jnp.dot(h, w_ref[...], preferred_element_type=jnp.float32)
        o_ref[0] = res[:, :HF]
        o_ref[1] = res[:, HF:]

    return pl.pallas_call(
        body,
        grid=(N // BLK,),
        in_specs=[
            pl.BlockSpec((2, BLK, HF), lambda i: (0, i, 0)),
            pl.BlockSpec((2, BLK, HF), lambda i: (0, i, 0)),
            pl.BlockSpec((2, BLK, DF), lambda i: (0, i, 0)),
            pl.BlockSpec((1, F), lambda i: (0, 0)),
            pl.BlockSpec((F, F), lambda i: (0, 0)),
        ],
        out_specs=pl.BlockSpec((2, BLK, HF), lambda i: (0, i, 0)),
        out_shape=jax.ShapeDtypeStruct((2, N, HF), jnp.float32),
    )(p, g0, degp, b0, W1)


def _tc_layer2(q, g1, degp, b1):
    def body(q_ref, g_ref, dp_ref, b_ref, o_ref):
        d = _d_of(dp_ref)
        s = jnp.concatenate([q_ref[0] + g_ref[0], q_ref[1] + g_ref[1]],
                            axis=-1)
        o_ref[...] = jax.nn.sigmoid(d * s + b_ref[...])

    return pl.pallas_call(
        body,
        grid=(N // BLK,),
        in_specs=[
            pl.BlockSpec((2, BLK, HF), lambda i: (0, i, 0)),
            pl.BlockSpec((2, BLK, HF), lambda i: (0, i, 0)),
            pl.BlockSpec((2, BLK, DF), lambda i: (0, i, 0)),
            pl.BlockSpec((1, F), lambda i: (0, 0)),
        ],
        out_specs=pl.BlockSpec((BLK, F), lambda i: (i, 0)),
        out_shape=jax.ShapeDtypeStruct((N, F), jnp.float32),
    )(q, g1, degp, b1)


def kernel(x, edge_index, W0, b0, W1, b1):
    src = edge_index[0]
    dst = edge_index[1]
    dpad = EPAD - EPT
    dstp_deg = jnp.pad(dst.reshape(T, EPT), ((0, 0), (0, dpad)),
                       constant_values=N).reshape(NC, NS, NB, B)
    spad = EPAD2 - EPT2
    src_t = jnp.pad(src.reshape(NS, EPT2),
                    ((0, 0), (0, spad))).reshape(NS, NB2, B)
    srcp = jnp.stack([src_t, src_t + N], axis=0)
    dstp = jnp.pad(dst.reshape(NS, EPT2), ((0, 0), (0, spad)),
                   constant_values=N).reshape(NS, NB2, B)
    ones = jnp.ones((B, DF), jnp.float32)
    zdeg = jnp.zeros((ROWS_PT, DF), jnp.float32)
    zrow = jnp.zeros((CH, HF), jnp.float32)

    degp = _sc_degree_kernel()(dstp_deg, ones, zdeg)
    g0 = _tc_layer0(x, W0, degp)
    p = _sc_scatter_kernel()(g0.reshape(NC * N, HF), srcp, dstp, zrow)
    g1 = _tc_layer1(p, g0, degp, b0.reshape(1, F), W1)
    q = _sc_scatter_kernel()(g1.reshape(NC * N, HF), srcp, dstp, zrow)
    return _tc_layer2(q, g1, degp, b1.reshape(1, F))
